# baseline (speedup 1.0000x reference)
import functools

import jax
import jax.numpy as jnp
from jax import lax
from jax.experimental import pallas as pl
from jax.experimental.pallas import tpu as pltpu
from jax.experimental.pallas import tpu_sc as plsc


def _make_bag_sum(V, D, B, L):
    info = plsc.get_sparse_core_info()
    NC, NS, LANES = info.num_cores, info.num_subcores, info.num_lanes
    NW = NC * NS
    assert B % NW == 0
    bags_w = B // NW
    C = 16
    assert bags_w % C == 0 and (bags_w // C) % 2 == 0
    n_chunks = bags_w // C
    rows_c = C * L
    assert D % LANES == 0
    KV = D // LANES
    g_sizes = []
    off = 0
    while off < rows_c:
        g_sizes.append(min(128, rows_c - off))
        off += 128

    mesh = plsc.VectorSubcoreMesh(core_axis_name="c", subcore_axis_name="s")

    @functools.partial(
        pl.kernel,
        mesh=mesh,
        compiler_params=pltpu.CompilerParams(use_tc_tiling_on_sc=False),
        out_type=jax.ShapeDtypeStruct((B * D,), jnp.float32),
        scratch_types=[
            pltpu.VMEM((rows_c,), jnp.int32),
            pltpu.VMEM((rows_c,), jnp.int32),
            pltpu.VMEM((rows_c, D), jnp.float32),
            pltpu.VMEM((rows_c, D), jnp.float32),
            pltpu.VMEM((C * D,), jnp.float32),
            pltpu.SemaphoreType.DMA,
            pltpu.SemaphoreType.DMA,
            pltpu.SemaphoreType.DMA,
            pltpu.SemaphoreType.DMA,
        ],
    )
    def bag_sum(table_hbm, idx_hbm, out_hbm,
                idx_v0, idx_v1, rows_v0, rows_v1, acc_v,
                gsem0, gsem1, isem0, isem1):
        wid = lax.axis_index("s") * NC + lax.axis_index("c")
        w_base = wid * bags_w

        def idx_slice(ci):
            return idx_hbm.at[pl.ds((w_base + ci * C) * L, rows_c)]

        def fire_idx(ci, idx_v, isem):
            pltpu.async_copy(idx_slice(ci), idx_v, isem)

        def wait_idx(idx_v, isem):
            pltpu.make_async_copy(idx_slice(0), idx_v, isem).wait()

        def fire_gather(idx_v, rows_v, gsem):
            o = 0
            for g in g_sizes:
                pltpu.async_copy(table_hbm.at[idx_v.at[pl.ds(o, g)]],
                                 rows_v.at[pl.ds(o, g)], gsem)
                o += g

        def drain_gather(rows_v, gsem):
            pltpu.make_async_copy(
                table_hbm.at[pl.ds(0, rows_c)], rows_v, gsem).wait()

        def accum_out(ci, rows_v):
            bag0 = w_base + ci * C

            def bag_body(j, carry):
                r0 = j * L
                accs = [rows_v[r0, pl.ds(k * LANES, LANES)]
                        for k in range(KV)]
                for r in range(1, L):
                    for k in range(KV):
                        accs[k] = accs[k] + rows_v[r0 + r,
                                                   pl.ds(k * LANES, LANES)]
                for k in range(KV):
                    acc_v[pl.ds(j * D + k * LANES, LANES)] = accs[k]
                return carry

            lax.fori_loop(0, C, bag_body, 0)
            pltpu.sync_copy(acc_v, out_hbm.at[pl.ds(bag0 * D, C * D)])

        fire_idx(0, idx_v0, isem0)
        wait_idx(idx_v0, isem0)
        fire_gather(idx_v0, rows_v0, gsem0)
        fire_idx(1, idx_v1, isem1)

        def pair_body(p, carry):
            c0 = 2 * p
            wait_idx(idx_v1, isem1)
            fire_gather(idx_v1, rows_v1, gsem1)
            drain_gather(rows_v0, gsem0)

            @pl.when(c0 + 2 < n_chunks)
            def _():
                fire_idx(c0 + 2, idx_v0, isem0)

            accum_out(c0, rows_v0)

            @pl.when(c0 + 2 < n_chunks)
            def _():
                wait_idx(idx_v0, isem0)
                fire_gather(idx_v0, rows_v0, gsem0)

            drain_gather(rows_v1, gsem1)

            @pl.when(c0 + 3 < n_chunks)
            def _():
                fire_idx(c0 + 3, idx_v1, isem1)

            accum_out(c0 + 1, rows_v1)
            return carry

        lax.fori_loop(0, n_chunks // 2, pair_body, 0)

    return bag_sum


def _transpose_body(x_ref, eye_ref, o_ref):
    o_ref[...] = jax.lax.dot_general(
        x_ref[...], eye_ref[...],
        dimension_numbers=(((0,), (0,)), ((), ())),
        preferred_element_type=jnp.float32)


def _tc_transpose(tin, V, D):
    TB = 8192
    grid = (V + TB - 1) // TB
    eye = jnp.eye(D, dtype=jnp.float32)
    return pl.pallas_call(
        _transpose_body,
        grid=(grid,),
        in_specs=[pl.BlockSpec((D, TB), lambda i: (0, i)),
                  pl.BlockSpec((D, D), lambda i: (0, 0))],
        out_specs=pl.BlockSpec((TB, D), lambda i: (i, 0)),
        out_shape=jax.ShapeDtypeStruct((V, D), jnp.float32),
    )(tin, eye)


def _mlp_body(x_ref, w1_ref, b1_ref, w2_ref, b2_ref, o_ref):
    h = jnp.dot(x_ref[...], w1_ref[...], preferred_element_type=jnp.float32)
    h = jnp.maximum(h + b1_ref[...], 0.0)
    o_ref[...] = jnp.sum(h * w2_ref[...], axis=1, keepdims=True) + b2_ref[...]


def kernel(text, emb_table, W1, b1, W2, b2):
    B, L = text.shape
    V, D = emb_table.shape
    idx_flat = text.reshape(B * L).astype(jnp.int32)

    table_rm = _tc_transpose(jnp.swapaxes(emb_table, 0, 1), V, D)

    bag_sum = _make_bag_sum(V, D, B, L)
    pooled = bag_sum(table_rm, idx_flat).reshape(B, D)

    w1s = (W1.T / jnp.float32(L)).astype(jnp.float32)
    b1r = b1.reshape(1, D)
    w2r = W2.reshape(1, D)
    b2r = b2.reshape(1, 1)

    BLK = 2048
    out = pl.pallas_call(
        _mlp_body,
        grid=(B // BLK,),
        in_specs=[
            pl.BlockSpec((BLK, D), lambda i: (i, 0)),
            pl.BlockSpec((D, D), lambda i: (0, 0)),
            pl.BlockSpec((1, D), lambda i: (0, 0)),
            pl.BlockSpec((1, D), lambda i: (0, 0)),
            pl.BlockSpec((1, 1), lambda i: (0, 0)),
        ],
        out_specs=pl.BlockSpec((BLK, 1), lambda i: (i, 0)),
        out_shape=jax.ShapeDtypeStruct((B, 1), jnp.float32),
    )(pooled, w1s, b1r, w2r, b2r)
    return jnp.squeeze(out, axis=-1)


# speedup vs baseline: 1.1055x; 1.1055x over previous
import functools

import jax
import jax.numpy as jnp
from jax import lax
from jax.experimental import pallas as pl
from jax.experimental.pallas import tpu as pltpu
from jax.experimental.pallas import tpu_sc as plsc

_TB = 4096


def _transpose_pack_body(xlo_ref, xhi_ref, eye_ref, o_ref):
    dn = (((0,), (0,)), ((), ()))
    D = eye_ref.shape[0]
    o_ref[:, :D] = jax.lax.dot_general(
        xlo_ref[...], eye_ref[...], dimension_numbers=dn,
        preferred_element_type=jnp.float32)
    o_ref[:, D:] = jax.lax.dot_general(
        xhi_ref[...], eye_ref[...], dimension_numbers=dn,
        preferred_element_type=jnp.float32)


def _tc_transpose_pack(tlo, thi, D, H):
    grid = H // _TB
    eye = jnp.eye(D, dtype=jnp.float32)
    return pl.pallas_call(
        _transpose_pack_body,
        grid=(grid,),
        in_specs=[pl.BlockSpec((D, _TB), lambda i: (0, i)),
                  pl.BlockSpec((D, _TB), lambda i: (0, i)),
                  pl.BlockSpec((D, D), lambda i: (0, 0))],
        out_specs=pl.BlockSpec((_TB, 2 * D), lambda i: (i, 0)),
        out_shape=jax.ShapeDtypeStruct((H, 2 * D), jnp.float32),
    )(tlo, thi, eye)


def _make_bag_sum(V, D, B, L, H):
    info = plsc.get_sparse_core_info()
    NC, NS, LANES = info.num_cores, info.num_subcores, info.num_lanes
    NW = NC * NS
    assert B % NW == 0
    bags_w = B // NW
    C = 8
    assert bags_w % C == 0 and (bags_w // C) % 2 == 0
    n_chunks = bags_w // C
    rows_c = C * L
    assert D % LANES == 0 and rows_c % LANES == 0
    KV = D // LANES
    PD = 2 * D
    g_sizes = []
    off = 0
    while off < rows_c:
        g_sizes.append(min(128, rows_c - off))
        off += 128

    mesh = plsc.VectorSubcoreMesh(core_axis_name="c", subcore_axis_name="s")

    @functools.partial(
        pl.kernel,
        mesh=mesh,
        compiler_params=pltpu.CompilerParams(use_tc_tiling_on_sc=True),
        out_type=jax.ShapeDtypeStruct((B * D,), jnp.float32),
        scratch_types=[
            pltpu.VMEM((rows_c,), jnp.int32),
            pltpu.VMEM((rows_c,), jnp.int32),
            pltpu.VMEM((rows_c + LANES,), jnp.int32),
            pltpu.VMEM((rows_c + LANES,), jnp.int32),
            pltpu.VMEM((rows_c, PD), jnp.float32),
            pltpu.VMEM((rows_c, PD), jnp.float32),
            pltpu.VMEM((C * D,), jnp.float32),
            pltpu.SemaphoreType.DMA,
            pltpu.SemaphoreType.DMA,
            pltpu.SemaphoreType.DMA,
            pltpu.SemaphoreType.DMA,
        ],
    )
    def bag_sum(table_hbm, idx_hbm, out_hbm,
                idx_v0, idx_v1, pb_v0, pb_v1, rows_v0, rows_v1, acc_v,
                gsem0, gsem1, isem0, isem1):
        wid = lax.axis_index("s") * NC + lax.axis_index("c")
        w_base = wid * bags_w

        def idx_slice(ci):
            return idx_hbm.at[pl.ds((w_base + ci * C) * L, rows_c)]

        def fire_idx(ci, idx_v, isem):
            pltpu.async_copy(idx_slice(ci), idx_v, isem)

        def wait_idx_prep(idx_v, pb_v, isem):
            pltpu.make_async_copy(idx_slice(0), idx_v, isem).wait()

            def prep(i, carry):
                v = idx_v[pl.ds(i * LANES, LANES)]
                hi = v >= H
                idx_v[pl.ds(i * LANES, LANES)] = jnp.where(hi, v - H, v)
                pb_v[pl.ds(i * LANES, LANES)] = jnp.where(hi, D, 0)
                return carry

            lax.fori_loop(0, rows_c // LANES, prep, 0)

        def fire_gather(idx_v, rows_v, gsem):
            o = 0
            for g in g_sizes:
                pltpu.async_copy(table_hbm.at[idx_v.at[pl.ds(o, g)]],
                                 rows_v.at[pl.ds(o, g)], gsem)
                o += g

        def drain_gather(rows_v, gsem):
            pltpu.make_async_copy(
                table_hbm.at[pl.ds(0, rows_c)], rows_v, gsem).wait()

        def accum_out(ci, pb_v, rows_v):
            bag0 = w_base + ci * C

            def bag_body(j, carry):
                r0 = j * L
                pbs = [pb_v[pl.ds(r0 + m * LANES, LANES)]
                       for m in range((L + LANES - 1) // LANES)]

                def base(r):
                    return pbs[r // LANES][r % LANES]

                accs = [rows_v[r0, pl.ds(base(0) + k * LANES, LANES)]
                        for k in range(KV)]
                for r in range(1, L):
                    b = base(r)
                    for k in range(KV):
                        accs[k] = accs[k] + rows_v[r0 + r,
                                                   pl.ds(b + k * LANES, LANES)]
                for k in range(KV):
                    acc_v[pl.ds(j * D + k * LANES, LANES)] = accs[k]
                return carry

            lax.fori_loop(0, C, bag_body, 0)
            pltpu.sync_copy(acc_v, out_hbm.at[pl.ds(bag0 * D, C * D)])

        fire_idx(0, idx_v0, isem0)
        wait_idx_prep(idx_v0, pb_v0, isem0)
        fire_gather(idx_v0, rows_v0, gsem0)
        fire_idx(1, idx_v1, isem1)

        def pair_body(p, carry):
            c0 = 2 * p
            wait_idx_prep(idx_v1, pb_v1, isem1)
            fire_gather(idx_v1, rows_v1, gsem1)
            drain_gather(rows_v0, gsem0)

            @pl.when(c0 + 2 < n_chunks)
            def _():
                fire_idx(c0 + 2, idx_v0, isem0)

            accum_out(c0, pb_v0, rows_v0)

            @pl.when(c0 + 2 < n_chunks)
            def _():
                wait_idx_prep(idx_v0, pb_v0, isem0)
                fire_gather(idx_v0, rows_v0, gsem0)

            drain_gather(rows_v1, gsem1)

            @pl.when(c0 + 3 < n_chunks)
            def _():
                fire_idx(c0 + 3, idx_v1, isem1)

            accum_out(c0 + 1, pb_v1, rows_v1)
            return carry

        lax.fori_loop(0, n_chunks // 2, pair_body, 0)

    return bag_sum


def _mlp_body(x_ref, w1_ref, b1_ref, w2_ref, b2_ref, o_ref):
    h = jnp.dot(x_ref[...], w1_ref[...], preferred_element_type=jnp.float32)
    h = jnp.maximum(h + b1_ref[...], 0.0)
    o_ref[...] = jnp.sum(h * w2_ref[...], axis=1, keepdims=True) + b2_ref[...]


def kernel(text, emb_table, W1, b1, W2, b2):
    B, L = text.shape
    V, D = emb_table.shape
    idx_flat = text.reshape(B * L).astype(jnp.int32)

    H = ((V // 2 + _TB - 1) // _TB) * _TB

    tin = jnp.swapaxes(emb_table, 0, 1)
    tlo = lax.slice(tin, (0, 0), (D, H))
    thi = jnp.pad(lax.slice(tin, (0, H), (D, V)), ((0, 0), (0, 2 * H - V)))
    table_packed = _tc_transpose_pack(tlo, thi, D, H)

    bag_sum = _make_bag_sum(V, D, B, L, H)
    pooled = bag_sum(table_packed, idx_flat).reshape(B, D)

    w1s = (W1.T / jnp.float32(L)).astype(jnp.float32)
    b1r = b1.reshape(1, D)
    w2r = W2.reshape(1, D)
    b2r = b2.reshape(1, 1)

    BLK = 2048
    out = pl.pallas_call(
        _mlp_body,
        grid=(B // BLK,),
        in_specs=[
            pl.BlockSpec((BLK, D), lambda i: (i, 0)),
            pl.BlockSpec((D, D), lambda i: (0, 0)),
            pl.BlockSpec((1, D), lambda i: (0, 0)),
            pl.BlockSpec((1, D), lambda i: (0, 0)),
            pl.BlockSpec((1, 1), lambda i: (0, 0)),
        ],
        out_specs=pl.BlockSpec((BLK, 1), lambda i: (i, 0)),
        out_shape=jax.ShapeDtypeStruct((B, 1), jnp.float32),
    )(pooled, w1s, b1r, w2r, b2r)
    return jnp.squeeze(out, axis=-1)


# speedup vs baseline: 1.2429x; 1.1243x over previous
import functools

import jax
import jax.numpy as jnp
from jax import lax
from jax.experimental import pallas as pl
from jax.experimental.pallas import tpu as pltpu
from jax.experimental.pallas import tpu_sc as plsc

_TB = 4096


def _transpose_pack_body(xlo_ref, xhi_ref, eye_ref, o_ref):
    dn = (((0,), (0,)), ((), ()))
    D = eye_ref.shape[0]
    o_ref[:, :D] = jax.lax.dot_general(
        xlo_ref[...], eye_ref[...], dimension_numbers=dn,
        preferred_element_type=jnp.float32)
    o_ref[:, D:] = jax.lax.dot_general(
        xhi_ref[...], eye_ref[...], dimension_numbers=dn,
        preferred_element_type=jnp.float32)


def _tc_transpose_pack(tlo, thi, D, H):
    grid = H // _TB
    eye = jnp.eye(D, dtype=jnp.float32)
    return pl.pallas_call(
        _transpose_pack_body,
        grid=(grid,),
        in_specs=[pl.BlockSpec((D, _TB), lambda i: (0, i)),
                  pl.BlockSpec((D, _TB), lambda i: (0, i)),
                  pl.BlockSpec((D, D), lambda i: (0, 0))],
        out_specs=pl.BlockSpec((_TB, 2 * D), lambda i: (i, 0)),
        out_shape=jax.ShapeDtypeStruct((H, 2 * D), jnp.float32),
    )(tlo, thi, eye)


def _make_bag_sum(V, D, B, L, H):
    info = plsc.get_sparse_core_info()
    NC, NS, LANES = info.num_cores, info.num_subcores, info.num_lanes
    NW = NC * NS
    assert B % NW == 0
    bags_w = B // NW
    C = 8
    assert bags_w % C == 0 and (bags_w // C) % 2 == 0
    n_chunks = bags_w // C
    rows_c = C * L
    assert D % LANES == 0 and rows_c % LANES == 0
    KV = D // LANES
    PD = 2 * D
    g_sizes = []
    off = 0
    while off < rows_c:
        g_sizes.append(min(128, rows_c - off))
        off += 128

    mesh = plsc.VectorSubcoreMesh(core_axis_name="c", subcore_axis_name="s")

    @functools.partial(
        pl.kernel,
        mesh=mesh,
        compiler_params=pltpu.CompilerParams(use_tc_tiling_on_sc=True),
        out_type=jax.ShapeDtypeStruct((B * D,), jnp.float32),
        scratch_types=[
            pltpu.VMEM((rows_c,), jnp.int32),
            pltpu.VMEM((rows_c,), jnp.int32),
            pltpu.VMEM((rows_c + LANES,), jnp.int32),
            pltpu.VMEM((rows_c + LANES,), jnp.int32),
            pltpu.VMEM((rows_c, PD), jnp.float32),
            pltpu.VMEM((rows_c, PD), jnp.float32),
            pltpu.VMEM((C * D,), jnp.float32),
            pltpu.SemaphoreType.DMA,
            pltpu.SemaphoreType.DMA,
            pltpu.SemaphoreType.DMA,
            pltpu.SemaphoreType.DMA,
        ],
    )
    def bag_sum(table_hbm, idx_hbm, out_hbm,
                idx_v0, idx_v1, pb_v0, pb_v1, rows_v0, rows_v1, acc_v,
                gsem0, gsem1, isem0, isem1):
        wid = lax.axis_index("s") * NC + lax.axis_index("c")
        w_base = wid * bags_w

        def idx_slice(ci):
            return idx_hbm.at[pl.ds((w_base + ci * C) * L, rows_c)]

        def fire_idx(ci, idx_v, isem):
            pltpu.async_copy(idx_slice(ci), idx_v, isem)

        def wait_idx_prep(idx_v, pb_v, isem):
            pltpu.make_async_copy(idx_slice(0), idx_v, isem).wait()

            def prep(i, carry):
                v = idx_v[pl.ds(i * LANES, LANES)]
                hi = v >= H
                idx_v[pl.ds(i * LANES, LANES)] = jnp.where(hi, v - H, v)
                pb_v[pl.ds(i * LANES, LANES)] = jnp.where(hi, D, 0)
                return carry

            lax.fori_loop(0, rows_c // LANES, prep, 0)

        def fire_gather(idx_v, rows_v, gsem):
            o = 0
            for g in g_sizes:
                pltpu.async_copy(table_hbm.at[idx_v.at[pl.ds(o, g)]],
                                 rows_v.at[pl.ds(o, g)], gsem)
                o += g

        def drain_gather(rows_v, gsem):
            pltpu.make_async_copy(
                table_hbm.at[pl.ds(0, rows_c)], rows_v, gsem).wait()

        def accum_out(ci, pb_v, rows_v):
            bag0 = w_base + ci * C

            def bag_body(j, carry):
                r0 = j * L
                pbs = [pb_v[pl.ds(r0 + m * LANES, LANES)]
                       for m in range((L + LANES - 1) // LANES)]

                def base(r):
                    return pbs[r // LANES][r % LANES]

                accs = [rows_v[r0, pl.ds(base(0) + k * LANES, LANES)]
                        for k in range(KV)]
                for r in range(1, L):
                    b = base(r)
                    for k in range(KV):
                        accs[k] = accs[k] + rows_v[r0 + r,
                                                   pl.ds(b + k * LANES, LANES)]
                for k in range(KV):
                    acc_v[pl.ds(j * D + k * LANES, LANES)] = accs[k]
                return carry

            lax.fori_loop(0, C, bag_body, 0)
            pltpu.sync_copy(acc_v, out_hbm.at[pl.ds(bag0 * D, C * D)])

        fire_idx(0, idx_v0, isem0)
        wait_idx_prep(idx_v0, pb_v0, isem0)
        fire_gather(idx_v0, rows_v0, gsem0)
        fire_idx(1, idx_v1, isem1)

        def pair_body(p, carry):
            c0 = 2 * p
            wait_idx_prep(idx_v1, pb_v1, isem1)
            fire_gather(idx_v1, rows_v1, gsem1)
            drain_gather(rows_v0, gsem0)

            @pl.when(c0 + 2 < n_chunks)
            def _():
                fire_idx(c0 + 2, idx_v0, isem0)

            accum_out(c0, pb_v0, rows_v0)

            @pl.when(c0 + 2 < n_chunks)
            def _():
                wait_idx_prep(idx_v0, pb_v0, isem0)
                fire_gather(idx_v0, rows_v0, gsem0)

            drain_gather(rows_v1, gsem1)

            @pl.when(c0 + 3 < n_chunks)
            def _():
                fire_idx(c0 + 3, idx_v1, isem1)

            accum_out(c0 + 1, pb_v1, rows_v1)
            return carry

        lax.fori_loop(0, n_chunks // 2, pair_body, 0)

    return bag_sum


def _mlp_body(x_ref, w1_ref, b1_ref, w2_ref, b2_ref, o_ref):
    h = jnp.dot(x_ref[...], w1_ref[...], preferred_element_type=jnp.float32)
    h = jnp.maximum(h + b1_ref[...], 0.0)
    o_ref[...] = jnp.sum(h * w2_ref[...], axis=1, keepdims=True) + b2_ref[...]


def kernel(text, emb_table, W1, b1, W2, b2):
    B, L = text.shape
    V, D = emb_table.shape
    idx_flat = text.reshape(B * L).astype(jnp.int32)

    H = ((V // 2 + _TB - 1) // _TB) * _TB

    tin = jnp.swapaxes(emb_table, 0, 1)
    thi = jnp.pad(lax.slice(tin, (0, H), (D, V)), ((0, 0), (0, 2 * H - V)))
    table_packed = _tc_transpose_pack(tin, thi, D, H)

    bag_sum = _make_bag_sum(V, D, B, L, H)
    pooled = bag_sum(table_packed, idx_flat).reshape(B, D)

    w1s = (W1.T / jnp.float32(L)).astype(jnp.float32)
    b1r = b1.reshape(1, D)
    w2r = W2.reshape(1, D)
    b2r = b2.reshape(1, 1)

    BLK = 2048
    out = pl.pallas_call(
        _mlp_body,
        grid=(B // BLK,),
        in_specs=[
            pl.BlockSpec((BLK, D), lambda i: (i, 0)),
            pl.BlockSpec((D, D), lambda i: (0, 0)),
            pl.BlockSpec((1, D), lambda i: (0, 0)),
            pl.BlockSpec((1, D), lambda i: (0, 0)),
            pl.BlockSpec((1, 1), lambda i: (0, 0)),
        ],
        out_specs=pl.BlockSpec((BLK, 1), lambda i: (i, 0)),
        out_shape=jax.ShapeDtypeStruct((B, 1), jnp.float32),
    )(pooled, w1s, b1r, w2r, b2r)
    return jnp.squeeze(out, axis=-1)
